# Initial kernel scaffold; baseline (speedup 1.0000x reference)
import functools

import jax
import jax.numpy as jnp
import numpy as np
from jax import lax
from jax.experimental import pallas as pl
from jax.experimental.pallas import tpu as pltpu
from jax.experimental.pallas import tpu_sc as plsc

_ROWS = 128
_COLS = 32768
_K = 64
_BETA = np.float32(1000000.0)
_L = 16
_NV = _COLS // _L
_HBITS = 12
_HSIZE = 1 << _HBITS
_CAP = 2048
_NC = 2
_NS = 16
_NW = _NC * _NS
_RPW = _ROWS // _NW


def _monokey(v):
  bi = lax.bitcast_convert_type(v, jnp.int32)
  bu = lax.bitcast_convert_type(v, jnp.uint32)
  flip = jnp.where(bi < 0, jnp.uint32(0xFFFFFFFF), jnp.uint32(0x80000000))
  return bu ^ flip


def _body(x_hbm, out_hbm, row_v, outrow_v, hist_v, candk_v, candv_v,
          candi_v, topi_v):
  wid = lax.axis_index("s") * _NC + lax.axis_index("c")
  lane = lax.iota(jnp.int32, _L)
  ones_i = jnp.ones((_L,), jnp.int32)
  zeros_i = jnp.zeros((_L,), jnp.int32)
  beta_vec = jnp.full((_L,), _BETA, jnp.float32)

  def fill(i, c):
    outrow_v[pl.ds(i * _L, _L)] = beta_vec
    return c
  lax.fori_loop(0, _NV, fill, 0)

  def do_row(w, c0):
    row = wid * _RPW + w

    pltpu.sync_copy(x_hbm.at[row], row_v)

    def clr(i, c):
      hist_v[pl.ds(i * _L, _L)] = zeros_i
      return c
    lax.fori_loop(0, _HSIZE // _L, clr, 0)

    def p1(i, c):
      v = row_v[pl.ds(i * _L, _L)]
      key = _monokey(v)
      b = lax.convert_element_type(key >> jnp.uint32(32 - _HBITS), jnp.int32)
      plsc.addupdate_scatter(hist_v, [b], ones_i)
      return c
    lax.fori_loop(0, _NV, p1, 0)

    def hscan(j, carry):
      tot_above, ntrue = carry
      h = hist_v[pl.ds(_HSIZE - _L * (j + 1), _L)]
      incl = plsc.cumsum(h)
      tot_this = jnp.max(incl)
      cumtop = (tot_above + tot_this) - incl + h
      cond = cumtop >= _K
      n = plsc.all_reduce_population_count(cond)
      return tot_above + tot_this, ntrue + jnp.max(n)
    _, ntrue = lax.fori_loop(0, _HSIZE // _L, hscan,
                             (jnp.int32(0), jnp.int32(0)))
    bkt = ntrue - 1
    a_key = lax.convert_element_type(bkt, jnp.uint32) << jnp.uint32(32 - _HBITS)
    a_bits = jnp.where(
        a_key >= jnp.uint32(0x80000000),
        a_key ^ jnp.uint32(0x80000000),
        ~a_key)
    t_a = lax.bitcast_convert_type(a_bits, jnp.float32)

    def p2(i, off_vec):
      v = row_v[pl.ds(i * _L, _L)]
      m = v >= t_a
      def hit(off_vec):
        key = _monokey(v)
        mi = lax.convert_element_type(m, jnp.int32)
        pos = jnp.minimum(off_vec + plsc.cumsum(mi) - mi,
                          jnp.int32(_CAP - 1))
        plsc.store_scatter(candk_v, [pos],
                           lax.bitcast_convert_type(key, jnp.int32), mask=m)
        plsc.store_scatter(candv_v, [pos], v, mask=m)
        plsc.store_scatter(candi_v, [pos], i * _L + lane, mask=m)
        return off_vec + plsc.all_reduce_population_count(m)
      return lax.cond(jnp.any(m), hit, lambda o: o, off_vec)
    off_vec = lax.fori_loop(0, _NV, p2, zeros_i)
    cnum = jnp.max(off_vec)
    nv_c = (cnum + _L - 1) // _L

    def count_ge(p_test):
      def cb(q, acc):
        gm = (q * _L + lane) < cnum
        key = lax.bitcast_convert_type(candk_v[pl.ds(q * _L, _L)], jnp.uint32)
        m = (key >= p_test) & gm
        return acc + plsc.all_reduce_population_count(m)
      return jnp.max(lax.fori_loop(0, nv_c, cb, zeros_i))

    p_cur = a_key
    for bit in range(32 - _HBITS - 1, -1, -1):
      p_try = p_cur | (jnp.uint32(1) << jnp.uint32(bit))
      cnt = count_ge(p_try)
      p_cur = jnp.where(cnt >= _K, p_try, p_cur)
    t_key = p_cur

    def gcb(q, acc):
      gm = (q * _L + lane) < cnum
      key = lax.bitcast_convert_type(candk_v[pl.ds(q * _L, _L)], jnp.uint32)
      m = (key > t_key) & gm
      return acc + plsc.all_reduce_population_count(m)
    g = jnp.max(lax.fori_loop(0, nv_c, gcb, zeros_i))
    d = _K - g

    def fin(q, carry):
      eqrun, keeprun = carry
      gm = (q * _L + lane) < cnum
      key = lax.bitcast_convert_type(candk_v[pl.ds(q * _L, _L)], jnp.uint32)
      gt = (key > t_key) & gm
      eq = (key == t_key) & gm
      eqi = lax.convert_element_type(eq, jnp.int32)
      eq_pre = eqrun + plsc.cumsum(eqi) - eqi
      keep = gt | (eq & (eq_pre < d))
      ki = lax.convert_element_type(keep, jnp.int32)
      pos = keeprun + plsc.cumsum(ki) - ki
      vals = candv_v[pl.ds(q * _L, _L)]
      cols = candi_v[pl.ds(q * _L, _L)]
      plsc.store_scatter(outrow_v, [cols], vals, mask=keep)
      plsc.store_scatter(topi_v, [pos], cols, mask=keep)
      return (eqrun + plsc.all_reduce_population_count(eq),
              keeprun + plsc.all_reduce_population_count(keep))
    lax.fori_loop(0, nv_c, fin, (zeros_i, zeros_i))

    pltpu.sync_copy(outrow_v, out_hbm.at[row])

    for j in range(_K // _L):
      cols = topi_v[pl.ds(j * _L, _L)]
      plsc.store_scatter(outrow_v, [cols], beta_vec)
    return c0

  lax.fori_loop(0, _RPW, do_row, 0)


@jax.jit
def kernel(x):
  mesh = plsc.VectorSubcoreMesh(
      core_axis_name="c", subcore_axis_name="s",
      num_cores=_NC, num_subcores=_NS)
  f = pl.kernel(
      _body,
      out_type=jax.ShapeDtypeStruct((_ROWS, _COLS), jnp.float32),
      mesh=mesh,
      scratch_types=[
          pltpu.VMEM((_COLS,), jnp.float32),
          pltpu.VMEM((_COLS,), jnp.float32),
          pltpu.VMEM((_HSIZE,), jnp.int32),
          pltpu.VMEM((_CAP,), jnp.int32),
          pltpu.VMEM((_CAP,), jnp.float32),
          pltpu.VMEM((_CAP,), jnp.int32),
          pltpu.VMEM((_K,), jnp.int32),
      ],
  )
  return f(x)


# speedup vs baseline: 2.4054x; 2.4054x over previous
import functools

import jax
import jax.numpy as jnp
import numpy as np
from jax import lax
from jax.experimental import pallas as pl
from jax.experimental.pallas import tpu as pltpu
from jax.experimental.pallas import tpu_sc as plsc

_ROWS = 128
_COLS = 32768
_K = 64
_BETA = np.float32(1000000.0)
_L = 16
_NV = _COLS // _L
_HBITS = 12
_HSIZE = 1 << _HBITS
_CAP = 2048
_NC = 2
_NS = 16
_NW = _NC * _NS
_RPW = _ROWS // _NW


def _monokey(v):
  bi = lax.bitcast_convert_type(v, jnp.int32)
  bu = lax.bitcast_convert_type(v, jnp.uint32)
  flip = jnp.where(bi < 0, jnp.uint32(0xFFFFFFFF), jnp.uint32(0x80000000))
  return bu ^ flip


def _body(x_hbm, out_hbm, row_v, outrow_v, hist_v, candk_v, candv_v,
          candi_v, topi_v):
  wid = lax.axis_index("s") * _NC + lax.axis_index("c")
  lane = lax.iota(jnp.int32, _L)
  ones_i = jnp.ones((_L,), jnp.int32)
  zeros_i = jnp.zeros((_L,), jnp.int32)
  beta_vec = jnp.full((_L,), _BETA, jnp.float32)

  def fill(i, c):
    outrow_v[pl.ds(i * _L, _L)] = beta_vec
    return c
  lax.fori_loop(0, _NV, fill, 0)

  def do_row(w, c0):
    row = wid * _RPW + w

    pltpu.sync_copy(x_hbm.at[row], row_v)

    def clr(i, c):
      hist_v[pl.ds(i * _L, _L)] = zeros_i
      return c
    lax.fori_loop(0, _HSIZE // _L, clr, 0)

    def p1(i, c):
      v = row_v[pl.ds(i * _L, _L)]
      key = _monokey(v)
      b = lax.convert_element_type(key >> jnp.uint32(32 - _HBITS), jnp.int32)
      plsc.addupdate_scatter(hist_v, [b], ones_i)
      return c
    lax.fori_loop(0, _NV, p1, 0)

    def hscan(j, carry):
      tot_above, ntrue = carry
      h = hist_v[pl.ds(_HSIZE - _L * (j + 1), _L)]
      incl = plsc.cumsum(h)
      tot_this = jnp.max(incl)
      cumtop = (tot_above + tot_this) - incl + h
      cond = cumtop >= _K
      n = plsc.all_reduce_population_count(cond)
      return tot_above + tot_this, ntrue + jnp.max(n)
    _, ntrue = lax.fori_loop(0, _HSIZE // _L, hscan,
                             (jnp.int32(0), jnp.int32(0)))
    bkt = ntrue - 1
    a_key = lax.convert_element_type(bkt, jnp.uint32) << jnp.uint32(32 - _HBITS)
    a_bits = jnp.where(
        a_key >= jnp.uint32(0x80000000),
        a_key ^ jnp.uint32(0x80000000),
        ~a_key)
    t_a = lax.bitcast_convert_type(a_bits, jnp.float32)

    def p2(i, off_vec):
      v = row_v[pl.ds(i * _L, _L)]
      m = v >= t_a
      def hit(off_vec):
        key = _monokey(v)
        mi = lax.convert_element_type(m, jnp.int32)
        pos = jnp.minimum(off_vec + plsc.cumsum(mi) - mi,
                          jnp.int32(_CAP - 1))
        plsc.store_scatter(candk_v, [pos],
                           lax.bitcast_convert_type(key, jnp.int32), mask=m)
        plsc.store_scatter(candv_v, [pos], v, mask=m)
        plsc.store_scatter(candi_v, [pos], i * _L + lane, mask=m)
        return off_vec + plsc.all_reduce_population_count(m)
      return lax.cond(jnp.any(m), hit, lambda o: o, off_vec)
    off_vec = lax.fori_loop(0, _NV, p2, zeros_i)
    cnum = jnp.max(off_vec)
    nv_c = (cnum + _L - 1) // _L

    def count_ge(p_test):
      def cb(q, acc):
        gm = (q * _L + lane) < cnum
        key = lax.bitcast_convert_type(candk_v[pl.ds(q * _L, _L)], jnp.uint32)
        m = (key >= p_test) & gm
        return acc + plsc.all_reduce_population_count(m)
      return jnp.max(lax.fori_loop(0, nv_c, cb, zeros_i))

    p_cur = a_key
    for bit in range(32 - _HBITS - 1, -1, -1):
      p_try = p_cur | (jnp.uint32(1) << jnp.uint32(bit))
      cnt = count_ge(p_try)
      p_cur = jnp.where(cnt >= _K, p_try, p_cur)
    t_key = p_cur

    def gcb(q, acc):
      gm = (q * _L + lane) < cnum
      key = lax.bitcast_convert_type(candk_v[pl.ds(q * _L, _L)], jnp.uint32)
      m = (key > t_key) & gm
      return acc + plsc.all_reduce_population_count(m)
    g = jnp.max(lax.fori_loop(0, nv_c, gcb, zeros_i))
    d = _K - g

    def fin(q, carry):
      eqrun, keeprun = carry
      gm = (q * _L + lane) < cnum
      key = lax.bitcast_convert_type(candk_v[pl.ds(q * _L, _L)], jnp.uint32)
      gt = (key > t_key) & gm
      eq = (key == t_key) & gm
      eqi = lax.convert_element_type(eq, jnp.int32)
      eq_pre = eqrun + plsc.cumsum(eqi) - eqi
      keep = gt | (eq & (eq_pre < d))
      ki = lax.convert_element_type(keep, jnp.int32)
      pos = keeprun + plsc.cumsum(ki) - ki
      vals = candv_v[pl.ds(q * _L, _L)]
      cols = candi_v[pl.ds(q * _L, _L)]
      plsc.store_scatter(outrow_v, [cols], vals, mask=keep)
      plsc.store_scatter(topi_v, [pos], cols, mask=keep)
      return (eqrun + plsc.all_reduce_population_count(eq),
              keeprun + plsc.all_reduce_population_count(keep))
    lax.fori_loop(0, nv_c, fin, (zeros_i, zeros_i))

    pltpu.sync_copy(outrow_v, out_hbm.at[row])

    for j in range(_K // _L):
      cols = topi_v[pl.ds(j * _L, _L)]
      plsc.store_scatter(outrow_v, [cols], beta_vec)
    return c0

  lax.fori_loop(0, _RPW, do_row, 0)


@jax.jit
def kernel(x):
  mesh = plsc.VectorSubcoreMesh(
      core_axis_name="c", subcore_axis_name="s",
      num_cores=_NC, num_subcores=_NS)
  f = pl.kernel(
      _body,
      out_type=jax.ShapeDtypeStruct((_ROWS, _COLS), jnp.float32),
      mesh=mesh,
      scratch_types=[
          pltpu.VMEM((_COLS,), jnp.float32),
          pltpu.VMEM((_COLS,), jnp.float32),
          pltpu.VMEM((_HSIZE,), jnp.int32),
          pltpu.VMEM((_CAP,), jnp.int32),
          pltpu.VMEM((_CAP,), jnp.float32),
          pltpu.VMEM((_CAP,), jnp.int32),
          pltpu.VMEM((_K,), jnp.int32),
      ],
      compiler_params=pltpu.CompilerParams(needs_layout_passes=False),
  )
  return f(x)


# speedup vs baseline: 7.0315x; 2.9232x over previous
import functools

import jax
import jax.numpy as jnp
import numpy as np
from jax import lax
from jax.experimental import pallas as pl
from jax.experimental.pallas import tpu as pltpu
from jax.experimental.pallas import tpu_sc as plsc

_ROWS = 128
_COLS = 32768
_K = 64
_BETA = np.float32(1000000.0)
_L = 16
_NV = _COLS // _L
_HBITS = 12
_HSIZE = 1 << _HBITS
_LOWBITS = 32 - _HBITS
_CAP = 2048
_G = 16
_NGROUP = _NV // _G
_NC = 2
_NS = 16
_NW = _NC * _NS
_RPW = _ROWS // _NW


def _monokey(v):
  bi = lax.bitcast_convert_type(v, jnp.int32)
  bu = lax.bitcast_convert_type(v, jnp.uint32)
  flip = lax.bitcast_convert_type(bi >> 31, jnp.uint32) | jnp.uint32(0x80000000)
  return bu ^ flip


def _invkey(key):
  bits = jnp.where(key >= jnp.uint32(0x80000000),
                   key ^ jnp.uint32(0x80000000), ~key)
  return lax.bitcast_convert_type(bits, jnp.float32)


def _body(x_hbm, out_hbm, rowa_v, rowb_v, outrow_v, hist_v, candk_v,
          candi_v, topi_v, insem, outsem):
  wid = lax.axis_index("s") * _NC + lax.axis_index("c")
  base_row = wid * _RPW
  lane = lax.iota(jnp.int32, _L)
  ones_i = jnp.ones((_L,), jnp.int32)
  zeros_i = jnp.zeros((_L,), jnp.int32)
  beta_vec = jnp.full((_L,), _BETA, jnp.float32)

  @plsc.parallel_loop(0, _NV, unroll=8)
  def _fill(i):
    outrow_v[pl.ds(i * _L, _L)] = beta_vec

  @plsc.parallel_loop(0, _HSIZE // _L, unroll=8)
  def _clr(i):
    hist_v[pl.ds(i * _L, _L)] = zeros_i

  in_cp = [None] * _RPW
  in_bufs = [rowa_v if w % 2 == 0 else rowb_v for w in range(_RPW)]
  in_cp[0] = pltpu.async_copy(x_hbm.at[base_row], in_bufs[0], insem)
  out_cp = None

  for w in range(_RPW):
    row_v = in_bufs[w]
    in_cp[w].wait()
    if w + 1 < _RPW:
      in_cp[w + 1] = pltpu.async_copy(
          x_hbm.at[base_row + (w + 1)], in_bufs[w + 1], insem)

    @plsc.parallel_loop(0, _NV, unroll=8)
    def _p1(i):
      v = row_v[pl.ds(i * _L, _L)]
      b = lax.convert_element_type(
          _monokey(v) >> jnp.uint32(_LOWBITS), jnp.int32)
      plsc.addupdate_scatter(hist_v, [b], ones_i)

    def hscan(j, carry):
      tot_above, ntrue = carry
      off = _HSIZE - _L * (j + 1)
      h = hist_v[pl.ds(off, _L)]
      hist_v[pl.ds(off, _L)] = zeros_i
      incl = plsc.cumsum(h)
      tot_this = jnp.max(incl)
      cumtop = (tot_above + tot_this) - incl + h
      n = plsc.all_reduce_population_count(cumtop >= _K)
      return tot_above + tot_this, ntrue + jnp.max(n)
    _, ntrue = plsc.parallel_loop(
        0, _HSIZE // _L, unroll=4,
        carry=(jnp.int32(0), jnp.int32(0)))(hscan)
    a_key = lax.convert_element_type(ntrue - 1, jnp.uint32) << jnp.uint32(
        _LOWBITS)
    t_a = _invkey(a_key)

    def p2(g, off_vec):
      base = g * _G
      vs = [row_v[pl.ds((base + t) * _L, _L)] for t in range(_G)]
      ms = [v >= t_a for v in vs]
      m_any = ms[0]
      for t in range(1, _G):
        m_any = m_any | ms[t]

      def hit(off_vec):
        for t in range(_G):
          v = row_v[pl.ds((base + t) * _L, _L)]
          m = v >= t_a
          key = _monokey(v)
          mi = lax.convert_element_type(m, jnp.int32)
          pos = jnp.minimum(off_vec + plsc.cumsum(mi) - mi,
                            jnp.int32(_CAP - 17))
          plsc.store_scatter(candk_v, [pos],
                             lax.bitcast_convert_type(key, jnp.int32),
                             mask=m)
          plsc.store_scatter(candi_v, [pos], (base + t) * _L + lane, mask=m)
          off_vec = off_vec + plsc.all_reduce_population_count(m)
        return off_vec
      return lax.cond(jnp.any(m_any), hit, lambda o: o, off_vec)
    off_vec = lax.fori_loop(0, _NGROUP, p2, zeros_i)
    cnum = jnp.max(off_vec)
    plsc.store_scatter(candk_v, [jnp.minimum(off_vec + lane,
                                             jnp.int32(_CAP - 1))], zeros_i)
    nv_c = (cnum + _L - 1) // _L

    def count_ge(p_test):
      def cb(q, acc):
        key = lax.bitcast_convert_type(candk_v[pl.ds(q * _L, _L)], jnp.uint32)
        return acc + plsc.all_reduce_population_count(key >= p_test)
      return jnp.max(lax.fori_loop(0, nv_c, cb, zeros_i))

    p_cur = a_key
    for bit in range(_LOWBITS - 1, -1, -1):
      p_try = p_cur | (jnp.uint32(1) << jnp.uint32(bit))
      cnt = count_ge(p_try)
      p_cur = jnp.where(cnt >= _K, p_try, p_cur)
    t_key = p_cur

    g_cnt = count_ge(t_key + jnp.uint32(1))
    d = _K - g_cnt

    if out_cp is not None:
      out_cp.wait()
      for j in range(_K // _L):
        cols = topi_v[pl.ds(j * _L, _L)]
        plsc.store_scatter(outrow_v, [cols], beta_vec)

    def fin(q, carry):
      eqrun, keeprun = carry
      key = lax.bitcast_convert_type(candk_v[pl.ds(q * _L, _L)], jnp.uint32)
      gt = key > t_key
      eq = key == t_key
      eqi = lax.convert_element_type(eq, jnp.int32)
      eq_pre = eqrun + plsc.cumsum(eqi) - eqi
      keep = gt | (eq & (eq_pre < d))
      ki = lax.convert_element_type(keep, jnp.int32)
      pos = keeprun + plsc.cumsum(ki) - ki
      cols = candi_v[pl.ds(q * _L, _L)]
      plsc.store_scatter(outrow_v, [cols], _invkey(key), mask=keep)
      plsc.store_scatter(topi_v, [pos], cols, mask=keep)
      return (eqrun + plsc.all_reduce_population_count(eq),
              keeprun + plsc.all_reduce_population_count(keep))
    lax.fori_loop(0, nv_c, fin, (zeros_i, zeros_i))

    out_cp = pltpu.async_copy(outrow_v, out_hbm.at[base_row + w], outsem)

  out_cp.wait()


@jax.jit
def kernel(x):
  mesh = plsc.VectorSubcoreMesh(
      core_axis_name="c", subcore_axis_name="s",
      num_cores=_NC, num_subcores=_NS)
  f = pl.kernel(
      _body,
      out_type=jax.ShapeDtypeStruct((_ROWS, _COLS), jnp.float32),
      mesh=mesh,
      scratch_types=[
          pltpu.VMEM((_COLS,), jnp.float32),
          pltpu.VMEM((_COLS,), jnp.float32),
          pltpu.VMEM((_COLS,), jnp.float32),
          pltpu.VMEM((_HSIZE,), jnp.int32),
          pltpu.VMEM((_CAP,), jnp.int32),
          pltpu.VMEM((_CAP,), jnp.int32),
          pltpu.VMEM((_K,), jnp.int32),
          pltpu.SemaphoreType.DMA,
          pltpu.SemaphoreType.DMA,
      ],
      compiler_params=pltpu.CompilerParams(needs_layout_passes=False),
  )
  return f(x)
